# baseline (speedup 1.0000x reference)
import functools

import jax
import jax.numpy as jnp
from jax import lax
from jax.experimental import pallas as pl
from jax.experimental.pallas import tpu as pltpu
from jax.experimental.pallas import tpu_sc as plsc

NC = 2
NS = 16
NW = NC * NS
L = 16
CB = 2048
UNR = 4
FPW = 2
PA = 50048
PB = 49920
TBASE = PA + PB
TPAD = 128


def _build(B, W, Fm1, V1):
  NCH = B // CB
  NSRC = Fm1 + 1
  assert W == FPW * NW and B % CB == 0 and CB % (L * UNR) == 0
  assert PA % 128 == 0 and PB % 128 == 0 and TBASE < V1 <= TBASE + TPAD

  mesh = plsc.VectorSubcoreMesh(core_axis_name="c", subcore_axis_name="s")

  @functools.partial(
      pl.kernel,
      out_type=jax.ShapeDtypeStruct((W, B), jnp.float32),
      mesh=mesh,
      scratch_types=[
          pltpu.VMEM((1, 1, PA), jnp.float32),
          pltpu.VMEM((1, 1, PB), jnp.float32),
          pltpu.VMEM((NSRC, 1, TPAD), jnp.float32),
          pltpu.VMEM((1, B), jnp.float32),
          pltpu.VMEM((1, CB), jnp.int32),
          pltpu.VMEM((1, CB), jnp.int32),
          pltpu.VMEM((Fm1, L), jnp.float32),
          pltpu.SemaphoreType.DMA,
          pltpu.SemaphoreType.DMA,
          pltpu.SemaphoreType.DMA,
          pltpu.SemaphoreType.DMA,
      ],
      compiler_params=pltpu.CompilerParams(
          use_tc_tiling_on_sc=True, needs_layout_passes=False),
  )
  def kern(e0r, tabs, tails, xT, zb, out,
           colA, colB, tail_v, out_v, idx0, idx1, zb_v,
           csemA, csemB, isem0, isem1):
    wid = lax.axis_index("s") * NC + lax.axis_index("c")
    zz = jnp.zeros((L,), jnp.int32)
    ibufs = (idx0, idx1)
    isems = (isem0, isem1)
    cols = (colA, colB)
    csems = (csemA, csemB)
    plen = (PA, PB)
    pbase = (0, PA)

    pltpu.sync_copy(zb, zb_v)

    def issue(src3d, srow, f, h):
      pltpu.async_copy(
          src3d.at[pl.ds(srow, 1), pl.ds(f, 1), pl.ds(pbase[h], plen[h])],
          cols[h], csems[h])

    def drain(h):
      pltpu.make_async_copy(
          tabs.at[pl.ds(0, 1), pl.ds(0, 1), pl.ds(0, plen[h])],
          cols[h], csems[h]).wait()

    def scan(ridx, h, w, init):
      base = pbase[h]
      ln = plen[h]
      col = cols[h]
      handles = {0: pltpu.async_copy(
          xT.at[pl.ds(ridx, 1), pl.ds(0, CB)], ibufs[0], isems[0])}
      for c in range(NCH):
        if c + 1 < NCH:
          nxt = (c + 1) % 2
          handles[c + 1] = pltpu.async_copy(
              xT.at[pl.ds(ridx, 1), pl.ds((c + 1) * CB, CB)],
              ibufs[nxt], isems[nxt])
        handles[c].wait()
        buf = ibufs[c % 2]

        def body(g, carry, c=c, buf=buf):
          for u in range(UNR):
            k = g * (L * UNR) + u * L
            v = buf[0, pl.ds(k, L)]
            voff = v - base
            vc = jnp.minimum(jnp.maximum(voff, 0), ln - 1)
            val = plsc.load_gather(col, [zz, zz, vc])
            if h == 0:
              msk = v < PA
            else:
              vt = jnp.minimum(jnp.maximum(v - TBASE, 0), TPAD - 1)
              tval = plsc.load_gather(
                  tail_v, [jnp.full((L,), ridx, jnp.int32), zz, vt])
              val = jnp.where(v < TBASE, val, tval)
              msk = v >= PA
            boff = c * CB + k
            wval = val if w is None else w * val
            if init:
              out_v[0, pl.ds(boff, L)] = jnp.where(msk, wval, 0.0)
            else:
              out_v[0, pl.ds(boff, L)] = (
                  out_v[0, pl.ds(boff, L)] + jnp.where(msk, wval, 0.0))
          return carry

        lax.fori_loop(0, CB // (L * UNR), body, 0)

    def weight(i):
      wrow = plsc.load_gather(
          zb_v, [jnp.full((L,), i, jnp.int32), lax.iota(jnp.int32, L)])
      return 0.5 * jnp.exp(wrow)

    def fbody(f_sel, carry):
      f = wid * FPW + f_sel
      pltpu.sync_copy(
          tails.at[pl.ds(0, NSRC), pl.ds(f, 1), pl.ds(0, TPAD)], tail_v)

      issue(e0r, 0, f, 0)
      issue(e0r, 0, f, 1)
      drain(0)
      scan(0, 0, None, True)
      issue(tabs, 0, f, 0)
      drain(1)
      scan(0, 1, None, False)
      issue(tabs, 0, f, 1)

      def tbody(i, carry2, f=f):
        w = weight(i)
        drain(0)
        scan(i + 1, 0, w, False)
        issue(tabs, i + 1, f, 0)
        drain(1)
        scan(i + 1, 1, w, False)
        issue(tabs, i + 1, f, 1)
        return carry2

      lax.fori_loop(0, Fm1 - 1, tbody, 0)

      wl = weight(Fm1 - 1)
      drain(0)
      scan(Fm1, 0, wl, False)
      drain(1)
      scan(Fm1, 1, wl, False)

      pltpu.sync_copy(out_v, out.at[pl.ds(f, 1), pl.ds(0, B)])
      return carry

    lax.fori_loop(0, FPW, fbody, 0)

  return kern


@jax.jit
def kernel(x, embed0, tables, zero):
  B, F = x.shape
  V1, W = embed0.shape
  Fm1 = F - 1

  tabs = tables.transpose(0, 2, 1)
  e0r = embed0.T.reshape(1, W, V1)
  xT = x.T
  zb = jnp.broadcast_to(zero[:, None], (Fm1, L))
  tails = jnp.concatenate([e0r, tabs], axis=0)[:, :, TBASE:]
  tails = jnp.pad(tails, ((0, 0), (0, 0), (0, TPAD - (V1 - TBASE))))

  kern = _build(B, W, Fm1, V1)
  outT = kern(e0r, tabs, tails, xT, zb)
  return outT.T
